# Initial kernel scaffold; baseline (speedup 1.0000x reference)
import jax
import jax.numpy as jnp
from jax.experimental import pallas as pl


def kernel(x, plane_coef, line_coef):
    raise NotImplementedError("write your pallas kernel here")


# speedup vs baseline: 392.5935x; 392.5935x over previous
import functools

import jax
import jax.numpy as jnp
from jax import lax
from jax.experimental import pallas as pl
from jax.experimental.pallas import tpu as pltpu
from jax.experimental.pallas import tpu_sc as plsc

_INFO = plsc.get_sparse_core_info()
_NC = _INFO.num_cores
_NS = _INFO.num_subcores
_NW = _NC * _NS
_L = _INFO.num_lanes

_PHASE_COLS = ((0, 1, 2), (0, 2, 1), (1, 2, 0))


def _make_sc_encoder(n, h, w, ncomp):
    pts = n // _NW
    m = 2048
    nchunk = pts // m
    nvec = m // _L
    plane_sz = ncomp * h * w
    scale_xy = float(w - 1)
    scale_z = float(h - 1)

    mesh = plsc.VectorSubcoreMesh(core_axis_name="c", subcore_axis_name="s")

    @functools.partial(
        pl.kernel,
        out_type=jax.ShapeDtypeStruct((3 * ncomp * n,), jnp.float32),
        mesh=mesh,
        compiler_params=pltpu.CompilerParams(needs_layout_passes=False),
        scratch_types=[
            pltpu.VMEM((plane_sz,), jnp.float32),
            pltpu.VMEM((ncomp * h,), jnp.float32),
            pltpu.VMEM((m,), jnp.float32),
            pltpu.VMEM((m,), jnp.float32),
            pltpu.VMEM((m,), jnp.float32),
            pltpu.VMEM((ncomp, m), jnp.float32),
        ],
    )
    def encoder(xt_hbm, ptab_hbm, ltab_hbm, out_hbm, tab_v, lt_v, xa_v, xb_v, xc_v, o_v):
        wid = lax.axis_index("s") * _NC + lax.axis_index("c")
        base0 = wid * pts

        for p in range(3):
            ca, cb, cz = _PHASE_COLS[p]
            pltpu.sync_copy(ptab_hbm.at[pl.ds(p * plane_sz, plane_sz)], tab_v)
            pltpu.sync_copy(ltab_hbm.at[pl.ds(p * ncomp * h, ncomp * h)], lt_v)

            def compute(i, _):
                s = pl.ds(i * _L, _L)
                gx = xa_v[s]
                gy = xb_v[s]
                gz = xc_v[s]
                ix = (gx + 1.0) * 0.5 * scale_xy
                iy = (gy + 1.0) * 0.5 * scale_xy
                iz = (gz + 1.0) * 0.5 * scale_z
                xi = jnp.minimum(jnp.maximum(ix.astype(jnp.int32), 0), w - 2)
                yi = jnp.minimum(jnp.maximum(iy.astype(jnp.int32), 0), h - 2)
                zi = jnp.minimum(jnp.maximum(iz.astype(jnp.int32), 0), h - 2)
                fx = ix - xi.astype(jnp.float32)
                fy = iy - yi.astype(jnp.float32)
                fz = iz - zi.astype(jnp.float32)
                f00 = yi * w + xi
                for c in range(ncomp):
                    i00 = f00 + (c * h * w)
                    g00 = plsc.load_gather(tab_v, [i00])
                    g01 = plsc.load_gather(tab_v, [i00 + 1])
                    g10 = plsc.load_gather(tab_v, [i00 + w])
                    g11 = plsc.load_gather(tab_v, [i00 + (w + 1)])
                    px0 = g00 + fx * (g01 - g00)
                    px1 = g10 + fx * (g11 - g10)
                    pv = px0 + fy * (px1 - px0)
                    li = zi + (c * h)
                    l0 = plsc.load_gather(lt_v, [li])
                    l1 = plsc.load_gather(lt_v, [li + 1])
                    lv = l0 + fz * (l1 - l0)
                    o_v[c, s] = pv * lv
                return 0

            def chunk_body(ch, _):
                gbase = base0 + ch * m
                pltpu.sync_copy(xt_hbm.at[pl.ds(ca * n + gbase, m)], xa_v)
                pltpu.sync_copy(xt_hbm.at[pl.ds(cb * n + gbase, m)], xb_v)
                pltpu.sync_copy(xt_hbm.at[pl.ds(cz * n + gbase, m)], xc_v)
                lax.fori_loop(0, nvec, compute, 0, unroll=False)
                for c in range(ncomp):
                    pltpu.sync_copy(
                        o_v.at[c], out_hbm.at[pl.ds((p * ncomp + c) * n + gbase, m)]
                    )
                return 0

            lax.fori_loop(0, nchunk, chunk_body, 0, unroll=False)

    return encoder


@jax.jit
def kernel(x, plane_coef, line_coef):
    n = x.shape[0]
    nplane, _, h, w = plane_coef.shape
    ncomp = 4
    xt = x.T.reshape(-1)
    ptab = plane_coef[:, :ncomp].reshape(-1)
    ltab = line_coef[:, :ncomp, :, 0].reshape(-1)
    flat = _make_sc_encoder(n, h, w, ncomp)(xt, ptab, ltab)
    return flat.reshape(3 * ncomp, n).T


# speedup vs baseline: 645.3215x; 1.6437x over previous
import functools

import jax
import jax.numpy as jnp
from jax import lax
from jax.experimental import pallas as pl
from jax.experimental.pallas import tpu as pltpu
from jax.experimental.pallas import tpu_sc as plsc

_INFO = plsc.get_sparse_core_info()
_NC = _INFO.num_cores
_NS = _INFO.num_subcores
_NW = _NC * _NS
_L = _INFO.num_lanes

_PHASE_COLS = ((0, 1, 2), (0, 2, 1), (1, 2, 0))


def _make_sc_encoder(n, h, w, ncomp):
    pts = n // _NW
    m = 2048
    nchunk = pts // m
    nvec = m // _L
    plane_sz = ncomp * h * w
    scale_xy = float(w - 1)
    scale_z = float(h - 1)

    mesh = plsc.VectorSubcoreMesh(core_axis_name="c", subcore_axis_name="s")

    @functools.partial(
        pl.kernel,
        out_type=jax.ShapeDtypeStruct((3 * ncomp * n,), jnp.float32),
        mesh=mesh,
        compiler_params=pltpu.CompilerParams(needs_layout_passes=False),
        scratch_types=[
            pltpu.VMEM((plane_sz,), jnp.float32),
            pltpu.VMEM((ncomp * h,), jnp.float32),
            pltpu.VMEM((pts,), jnp.float32),
            pltpu.VMEM((pts,), jnp.float32),
            pltpu.VMEM((pts,), jnp.float32),
            pltpu.VMEM((ncomp, m), jnp.float32),
        ],
    )
    def encoder(xt_hbm, ptab_hbm, ltab_hbm, out_hbm, tab_v, lt_v, x0_v, x1_v, x2_v, o_v):
        wid = lax.axis_index("s") * _NC + lax.axis_index("c")
        base0 = wid * pts

        cols = (x0_v, x1_v, x2_v)
        for c in range(3):
            pltpu.sync_copy(xt_hbm.at[pl.ds(c * n + base0, pts)], cols[c])

        for p in range(3):
            ca, cb, cz = _PHASE_COLS[p]
            xa_v, xb_v, xc_v = cols[ca], cols[cb], cols[cz]
            pltpu.sync_copy(ptab_hbm.at[pl.ds(p * plane_sz, plane_sz)], tab_v)
            pltpu.sync_copy(ltab_hbm.at[pl.ds(p * ncomp * h, ncomp * h)], lt_v)

            def chunk_body(ch, _):
                cbase = ch * m

                @plsc.parallel_loop(0, nvec, unroll=2)
                def compute(i):
                    s = pl.ds(i * _L, _L)
                    sx = pl.ds(cbase + i * _L, _L)
                    gx = xa_v[sx]
                    gy = xb_v[sx]
                    gz = xc_v[sx]
                    ix = (gx + 1.0) * 0.5 * scale_xy
                    iy = (gy + 1.0) * 0.5 * scale_xy
                    iz = (gz + 1.0) * 0.5 * scale_z
                    xi = jnp.minimum(jnp.maximum(ix.astype(jnp.int32), 0), w - 2)
                    yi = jnp.minimum(jnp.maximum(iy.astype(jnp.int32), 0), h - 2)
                    zi = jnp.minimum(jnp.maximum(iz.astype(jnp.int32), 0), h - 2)
                    fx = ix - xi.astype(jnp.float32)
                    fy = iy - yi.astype(jnp.float32)
                    fz = iz - zi.astype(jnp.float32)
                    f00 = yi * w + xi
                    for c in range(ncomp):
                        i00 = f00 + (c * h * w)
                        g00 = plsc.load_gather(tab_v, [i00])
                        g01 = plsc.load_gather(tab_v, [i00 + 1])
                        g10 = plsc.load_gather(tab_v, [i00 + w])
                        g11 = plsc.load_gather(tab_v, [i00 + (w + 1)])
                        px0 = g00 + fx * (g01 - g00)
                        px1 = g10 + fx * (g11 - g10)
                        pv = px0 + fy * (px1 - px0)
                        li = zi + (c * h)
                        l0 = plsc.load_gather(lt_v, [li])
                        l1 = plsc.load_gather(lt_v, [li + 1])
                        lv = l0 + fz * (l1 - l0)
                        o_v[c, s] = pv * lv

                gbase = base0 + cbase
                for c in range(ncomp):
                    pltpu.sync_copy(
                        o_v.at[c], out_hbm.at[pl.ds((p * ncomp + c) * n + gbase, m)]
                    )
                return 0

            lax.fori_loop(0, nchunk, chunk_body, 0, unroll=False)

    return encoder


@jax.jit
def kernel(x, plane_coef, line_coef):
    n = x.shape[0]
    nplane, _, h, w = plane_coef.shape
    ncomp = 4
    xt = x.T.reshape(-1)
    ptab = plane_coef[:, :ncomp].reshape(-1)
    ltab = line_coef[:, :ncomp, :, 0].reshape(-1)
    flat = _make_sc_encoder(n, h, w, ncomp)(xt, ptab, ltab)
    return flat.reshape(3 * ncomp, n).T


# speedup vs baseline: 697.3862x; 1.0807x over previous
import functools

import jax
import jax.numpy as jnp
from jax import lax
from jax.experimental import pallas as pl
from jax.experimental.pallas import tpu as pltpu
from jax.experimental.pallas import tpu_sc as plsc

_INFO = plsc.get_sparse_core_info()
_NC = _INFO.num_cores
_NS = _INFO.num_subcores
_NW = _NC * _NS
_L = _INFO.num_lanes

_PHASE_COLS = ((0, 1, 2), (0, 2, 1), (1, 2, 0))


def _make_sc_encoder(n, h, w, ncomp):
    pts = n // _NW
    m = 2048
    nchunk = pts // m
    nvec = m // _L
    plane_sz = ncomp * h * w
    scale_xy = float(w - 1)
    scale_z = float(h - 1)

    mesh = plsc.VectorSubcoreMesh(core_axis_name="c", subcore_axis_name="s")

    @functools.partial(
        pl.kernel,
        out_type=jax.ShapeDtypeStruct((3 * ncomp * n,), jnp.float32),
        mesh=mesh,
        compiler_params=pltpu.CompilerParams(needs_layout_passes=False),
        scratch_types=[
            pltpu.VMEM((plane_sz,), jnp.float32),
            pltpu.VMEM((ncomp * h,), jnp.float32),
            pltpu.VMEM((pts,), jnp.float32),
            pltpu.VMEM((pts,), jnp.float32),
            pltpu.VMEM((pts,), jnp.float32),
            pltpu.VMEM((ncomp, m), jnp.float32),
        ],
    )
    def encoder(xt_hbm, ptab_hbm, ltab_hbm, out_hbm, tab_v, lt_v, x0_v, x1_v, x2_v, o_v):
        wid = lax.axis_index("s") * _NC + lax.axis_index("c")
        base0 = wid * pts

        cols = (x0_v, x1_v, x2_v)
        for c in range(3):
            pltpu.sync_copy(xt_hbm.at[pl.ds(c * n + base0, pts)], cols[c])

        for p in range(3):
            ca, cb, cz = _PHASE_COLS[p]
            xa_v, xb_v, xc_v = cols[ca], cols[cb], cols[cz]
            pltpu.sync_copy(ptab_hbm.at[pl.ds(p * plane_sz, plane_sz)], tab_v)
            pltpu.sync_copy(ltab_hbm.at[pl.ds(p * ncomp * h, ncomp * h)], lt_v)

            def chunk_body(ch, _):
                cbase = ch * m

                @plsc.parallel_loop(0, nvec, unroll=4)
                def compute(i):
                    s = pl.ds(i * _L, _L)
                    sx = pl.ds(cbase + i * _L, _L)
                    gx = xa_v[sx]
                    gy = xb_v[sx]
                    gz = xc_v[sx]
                    ix = gx * (0.5 * scale_xy) + (0.5 * scale_xy)
                    iy = gy * (0.5 * scale_xy) + (0.5 * scale_xy)
                    iz = gz * (0.5 * scale_z) + (0.5 * scale_z)
                    xi = jnp.minimum(ix.astype(jnp.int32), w - 2)
                    yi = jnp.minimum(iy.astype(jnp.int32), h - 2)
                    zi = jnp.minimum(iz.astype(jnp.int32), h - 2)
                    fx = ix - xi.astype(jnp.float32)
                    fy = iy - yi.astype(jnp.float32)
                    fz = iz - zi.astype(jnp.float32)
                    f00 = yi * w + xi
                    for c in range(ncomp):
                        i00 = f00 + (c * h * w)
                        g00 = plsc.load_gather(tab_v, [i00])
                        g01 = plsc.load_gather(tab_v, [i00 + 1])
                        g10 = plsc.load_gather(tab_v, [i00 + w])
                        g11 = plsc.load_gather(tab_v, [i00 + (w + 1)])
                        px0 = g00 + fx * (g01 - g00)
                        px1 = g10 + fx * (g11 - g10)
                        pv = px0 + fy * (px1 - px0)
                        li = zi + (c * h)
                        l0 = plsc.load_gather(lt_v, [li])
                        l1 = plsc.load_gather(lt_v, [li + 1])
                        lv = l0 + fz * (l1 - l0)
                        o_v[c, s] = pv * lv

                gbase = base0 + cbase
                for c in range(ncomp):
                    pltpu.sync_copy(
                        o_v.at[c], out_hbm.at[pl.ds((p * ncomp + c) * n + gbase, m)]
                    )
                return 0

            lax.fori_loop(0, nchunk, chunk_body, 0, unroll=False)

    return encoder


@jax.jit
def kernel(x, plane_coef, line_coef):
    n = x.shape[0]
    nplane, _, h, w = plane_coef.shape
    ncomp = 4
    xt = x.T.reshape(-1)
    ptab = plane_coef[:, :ncomp].reshape(-1)
    ltab = line_coef[:, :ncomp, :, 0].reshape(-1)
    flat = _make_sc_encoder(n, h, w, ncomp)(xt, ptab, ltab)
    return flat.reshape(3 * ncomp, n).T
